# baseline (speedup 1.0000x reference)
import functools

import jax
import jax.numpy as jnp
from jax import lax
from jax.experimental import pallas as pl
from jax.experimental.pallas import tpu as pltpu
from jax.experimental.pallas import tpu_sc as plsc

D_MODEL = 64
SCALE = 8.0
CHUNK = 128
NBUF = 4


@functools.lru_cache(maxsize=None)
def _make_kernel(NB, NT):
    info = plsc.get_sparse_core_info()
    nc, ns = info.num_cores, info.num_subcores
    nw = nc * ns
    b_per_w = (NB // nw) * NT
    n_outer = NT // NBUF
    assert NB == nw * CHUNK and n_outer * NBUF == NT

    mesh = plsc.VectorSubcoreMesh(core_axis_name="c", subcore_axis_name="s")

    @functools.partial(
        pl.kernel,
        mesh=mesh,
        out_type=jax.ShapeDtypeStruct((NT, D_MODEL, NB), jnp.float32),
        compiler_params=pltpu.CompilerParams(
            use_tc_tiling_on_sc=True, needs_layout_passes=False
        ),
        scratch_types=(
            [pltpu.VMEM((b_per_w,), jnp.int32)]
            + [pltpu.VMEM((CHUNK,), jnp.int32) for _ in range(NBUF)]
            + [pltpu.VMEM((CHUNK,), jnp.int32) for _ in range(NBUF)]
            + [pltpu.VMEM((CHUNK, 128), jnp.float32) for _ in range(NBUF)]
            + [pltpu.VMEM((D_MODEL, CHUNK), jnp.float32) for _ in range(NBUF)]
            + [pltpu.SemaphoreType.DMA for _ in range(NBUF)]
        ),
    )
    def emb_kernel(x_hbm, lut_hbm, out_hbm, idx_v, *rest):
        ibufs = rest[:NBUF]
        pbufs = rest[NBUF:2 * NBUF]
        gbufs = rest[2 * NBUF:3 * NBUF]
        obufs = rest[3 * NBUF:4 * NBUF]
        sems = rest[4 * NBUF:]
        wid = lax.axis_index("s") * nc + lax.axis_index("c")
        base = wid * b_per_w

        pltpu.sync_copy(x_hbm.at[pl.ds(base, b_per_w)], idx_v)

        lanes = lax.iota(jnp.int32, 16)

        def start_gather(t, b):
            for l0 in range(CHUNK // 16):
                src = (l0 * 16 + lanes) * NT + t
                v = plsc.load_gather(idx_v, [src])
                ibufs[b][pl.ds(l0 * 16, 16)] = lax.shift_right_logical(v, 1)
                pbufs[b][pl.ds(l0 * 16, 16)] = v & 1
            pltpu.async_copy(lut_hbm.at[ibufs[b]], gbufs[b], sems[b])

        def wait_gather(b):
            pltpu.make_async_copy(lut_hbm.at[ibufs[b]], gbufs[b], sems[b]).wait()

        for b in range(NBUF):
            start_gather(b, b)

        def outer(o, carry):
            t0 = o * NBUF
            for b in range(NBUF):
                t = t0 + b
                wait_gather(b)

                gbuf, obuf, pbuf = gbufs[b], obufs[b], pbufs[b]

                def pack(d, c2, gbuf=gbuf, obuf=obuf, pbuf=pbuf):
                    for l0 in range(CHUNK // 16):
                        rows = l0 * 16 + lanes
                        cols = pbuf[pl.ds(l0 * 16, 16)] * 64 + d
                        obuf[d, pl.ds(l0 * 16, 16)] = plsc.load_gather(
                            gbuf, [rows, cols]
                        )
                    return c2

                lax.fori_loop(0, D_MODEL, pack, 0, unroll=2)

                @pl.when(t + NBUF < NT)
                def _():
                    start_gather(t + NBUF, b)

                pltpu.sync_copy(
                    obuf,
                    out_hbm.at[t, :, pl.ds(pl.multiple_of(wid * CHUNK, CHUNK), CHUNK)],
                )
            return carry

        lax.fori_loop(0, n_outer, outer, 0)

    return emb_kernel


def kernel(x, lut):
    NB, NT = x.shape
    xf = x.reshape(NB * NT).astype(jnp.int32)
    lutp = jnp.concatenate([lut[0::2], lut[1::2]], axis=1) * SCALE
    out = _make_kernel(NB, NT)(xf, lutp)
    return out.transpose(2, 0, 1)


# speedup vs baseline: 5.7037x; 5.7037x over previous
import functools

import jax
import jax.numpy as jnp
from jax import lax
from jax.experimental import pallas as pl
from jax.experimental.pallas import tpu as pltpu
from jax.experimental.pallas import tpu_sc as plsc

D_MODEL = 64
SCALE = 8.0
CHUNK = 128
NBUF = 4


@functools.lru_cache(maxsize=None)
def _make_kernel(NB, NT):
    info = plsc.get_sparse_core_info()
    nc, ns = info.num_cores, info.num_subcores
    nw = nc * ns
    b_per_w = (NB // nw) * NT
    n_outer = NT // NBUF
    assert NB == nw * CHUNK and n_outer * NBUF == NT

    mesh = plsc.VectorSubcoreMesh(core_axis_name="c", subcore_axis_name="s")

    @functools.partial(
        pl.kernel,
        mesh=mesh,
        out_type=jax.ShapeDtypeStruct((NT, D_MODEL, NB), jnp.float32),
        compiler_params=pltpu.CompilerParams(
            use_tc_tiling_on_sc=True, needs_layout_passes=False
        ),
        scratch_types=(
            [pltpu.VMEM((b_per_w,), jnp.int32)]
            + [pltpu.VMEM((CHUNK,), jnp.int32) for _ in range(NBUF)]
            + [pltpu.VMEM((CHUNK, 128), jnp.float32) for _ in range(NBUF)]
            + [pltpu.VMEM((D_MODEL, CHUNK), jnp.float32) for _ in range(NBUF)]
            + [pltpu.SemaphoreType.DMA for _ in range(NBUF)]
            + [pltpu.SemaphoreType.DMA for _ in range(NBUF)]
        ),
    )
    def emb_kernel(x_hbm, lut_hbm, out_hbm, idx_v, *rest):
        ibufs = rest[:NBUF]
        gbufs = rest[NBUF:2 * NBUF]
        obufs = rest[2 * NBUF:3 * NBUF]
        gsems = rest[3 * NBUF:4 * NBUF]
        osems = rest[4 * NBUF:]
        wid = lax.axis_index("s") * nc + lax.axis_index("c")
        base = wid * b_per_w

        pltpu.sync_copy(x_hbm.at[pl.ds(base, b_per_w)], idx_v)

        lanes = lax.iota(jnp.int32, 16)

        def start_gather(t, b):
            for l0 in range(CHUNK // 16):
                src = (l0 * 16 + lanes) * NT + t
                ibufs[b][pl.ds(l0 * 16, 16)] = plsc.load_gather(idx_v, [src])
            pltpu.async_copy(lut_hbm.at[ibufs[b]], gbufs[b], gsems[b])

        def wait_gather(b):
            pltpu.make_async_copy(lut_hbm.at[ibufs[b]], gbufs[b], gsems[b]).wait()

        def out_descs(t, b):
            return [
                pltpu.make_async_copy(
                    obufs[b].at[pl.ds(i * 8, 8), :],
                    out_hbm.at[
                        t,
                        pl.ds(i * 8, 8),
                        pl.ds(pl.multiple_of(wid * CHUNK, CHUNK), CHUNK),
                    ],
                    osems[b],
                )
                for i in range(D_MODEL // 8)
            ]

        for b in range(NBUF):
            start_gather(b, b)

        def outer(o, carry):
            t0 = o * NBUF
            for b in range(NBUF):
                t = t0 + b
                wait_gather(b)

                @pl.when(t >= NBUF)
                def _():
                    for d in out_descs(t - NBUF, b):
                        d.wait()

                gbuf, obuf = gbufs[b], obufs[b]

                def pack(d, c2, gbuf=gbuf, obuf=obuf):
                    cols = jnp.full((16,), 0, jnp.int32) + d
                    for l0 in range(CHUNK // 16):
                        rows = l0 * 16 + lanes
                        obuf[d, pl.ds(l0 * 16, 16)] = (
                            plsc.load_gather(gbuf, [rows, cols]) * SCALE
                        )
                    return c2

                lax.fori_loop(0, D_MODEL, pack, 0, unroll=2)

                @pl.when(t + NBUF < NT)
                def _():
                    start_gather(t + NBUF, b)

                for d in out_descs(t, b):
                    d.start()
            return carry

        lax.fori_loop(0, n_outer, outer, 0)

        for b in range(NBUF):
            for d in out_descs(NT - NBUF + b, (NT - NBUF + b) % NBUF):
                d.wait()

    return emb_kernel


def kernel(x, lut):
    NB, NT = x.shape
    xf = x.reshape(NB * NT).astype(jnp.int32)
    lutp = jnp.pad(lut, ((0, 0), (0, 128 - lut.shape[1])))
    out = _make_kernel(NB, NT)(xf, lutp)
    return out.transpose(2, 0, 1)


# speedup vs baseline: 9.4776x; 1.6617x over previous
import functools

import jax
import jax.numpy as jnp
from jax import lax
from jax.experimental import pallas as pl
from jax.experimental.pallas import tpu as pltpu
from jax.experimental.pallas import tpu_sc as plsc

D_MODEL = 64
SCALE = 8.0
CHUNK = 64
DEPTH = 4
NRING = 2 * DEPTH


@functools.lru_cache(maxsize=None)
def _make_kernel(B):
    info = plsc.get_sparse_core_info()
    nc, ns = info.num_cores, info.num_subcores
    nw = nc * ns
    b_per_w = B // nw
    n_chunks = b_per_w // CHUNK
    n_outer = n_chunks // NRING
    assert b_per_w * nw == B and n_outer * NRING == n_chunks

    mesh = plsc.VectorSubcoreMesh(core_axis_name="c", subcore_axis_name="s")

    @functools.partial(
        pl.kernel,
        mesh=mesh,
        out_type=jax.ShapeDtypeStruct((B, 128), jnp.float32),
        compiler_params=pltpu.CompilerParams(
            use_tc_tiling_on_sc=True, needs_layout_passes=False
        ),
        scratch_types=(
            [pltpu.VMEM((b_per_w,), jnp.int32)]
            + [pltpu.VMEM((CHUNK, 128), jnp.float32) for _ in range(NRING)]
            + [pltpu.SemaphoreType.DMA for _ in range(NRING)]
            + [pltpu.SemaphoreType.DMA for _ in range(NRING)]
        ),
    )
    def emb_kernel(x_hbm, lut_hbm, out_hbm, idx_v, *rest):
        gbufs = rest[:NRING]
        gsems = rest[NRING:2 * NRING]
        osems = rest[2 * NRING:]
        wid = lax.axis_index("s") * nc + lax.axis_index("c")
        base = wid * b_per_w

        pltpu.sync_copy(x_hbm.at[pl.ds(base, b_per_w)], idx_v)

        def gather_desc(g, b):
            idx_slice = idx_v.at[pl.ds(pl.multiple_of(g * CHUNK, CHUNK), CHUNK)]
            return pltpu.make_async_copy(lut_hbm.at[idx_slice], gbufs[b], gsems[b])

        def write_desc(g, b):
            dst = out_hbm.at[pl.ds(base + pl.multiple_of(g * CHUNK, CHUNK), CHUNK)]
            return pltpu.make_async_copy(gbufs[b], dst, osems[b])

        for b in range(DEPTH):
            gather_desc(b, b).start()

        def outer(o, carry):
            g0 = o * NRING
            for s in range(NRING):
                g = g0 + s
                gather_desc(g, s).wait()
                write_desc(g, s).start()

                s2 = (s + DEPTH) % NRING

                @pl.when(g >= DEPTH)
                def _():
                    write_desc(g - DEPTH, s2).wait()

                @pl.when(g + DEPTH < n_chunks)
                def _():
                    gather_desc(g + DEPTH, s2).start()
            return carry

        lax.fori_loop(0, n_outer, outer, 0)

        for g in range(n_chunks - DEPTH, n_chunks):
            write_desc(g, g % NRING).wait()

    return emb_kernel


def kernel(x, lut):
    NB, NT = x.shape
    B = NB * NT
    xf = x.reshape(B).astype(jnp.int32)
    lutp = jnp.pad(lut, ((0, 0), (0, 128 - lut.shape[1])))
    out = _make_kernel(B)(xf, lutp)
    return out.reshape(NB, NT, 128)[:, :, :D_MODEL] * SCALE
